# Initial kernel scaffold; baseline (speedup 1.0000x reference)
import functools

import jax
import jax.numpy as jnp
from jax import lax
from jax.experimental import pallas as pl
from jax.experimental.pallas import tpu as pltpu
from jax.experimental.pallas import tpu_sc as plsc

N_NODES_ = 10000
N_EDGES_ = 320000
D_ = 128

NUM_CORES = 2
NUM_SUBCORES = 16
NUM_TILES = NUM_CORES * NUM_SUBCORES
EDGES_PER_TILE = N_EDGES_ // NUM_TILES
CHUNK = 80
CHUNKS_PER_TILE = EDGES_PER_TILE // CHUNK
ROWS_PER_TILE = N_NODES_ // NUM_SUBCORES
ZROWS = 125


def _sc_body(x_hbm, row_hbm, col_hbm, out_hbm, colv, rowv, rows_v, zbuf, acc, gsem):
    cid = lax.axis_index("c")
    sid = lax.axis_index("s")
    wid = sid * NUM_CORES + cid

    z = jnp.zeros((16,), jnp.float32)

    def _zero_body(i, _):
        for j in range(D_ // 16):
            zbuf[i, pl.ds(j * 16, 16)] = z
        return 0

    lax.fori_loop(0, ZROWS, _zero_body, 0)
    row_base = sid * ROWS_PER_TILE
    for k in range(ROWS_PER_TILE // ZROWS):
        pltpu.sync_copy(zbuf, acc.at[pl.ds(row_base + k * ZROWS, ZROWS)])
    plsc.subcore_barrier()

    edge_base = wid * EDGES_PER_TILE

    def _chunk_body(ci, _):
        base = edge_base + ci * CHUNK
        pltpu.sync_copy(col_hbm.at[pl.ds(base, CHUNK)], colv)
        pltpu.sync_copy(row_hbm.at[pl.ds(base, CHUNK)], rowv)
        pltpu.async_copy(x_hbm.at[colv], rows_v, gsem).wait()
        pltpu.sync_copy(rows_v, acc.at[rowv], add=True)
        return 0

    lax.fori_loop(0, CHUNKS_PER_TILE, _chunk_body, 0)
    plsc.subcore_barrier()

    pltpu.sync_copy(
        acc.at[pl.ds(row_base, ROWS_PER_TILE)],
        out_hbm.at[cid, pl.ds(row_base, ROWS_PER_TILE)],
    )


@jax.jit
def _sc_scatter(x, row, col):
    mesh = plsc.VectorSubcoreMesh(core_axis_name="c", subcore_axis_name="s")
    return pl.kernel(
        _sc_body,
        out_type=jax.ShapeDtypeStruct((NUM_CORES, N_NODES_, D_), jnp.float32),
        mesh=mesh,
        scratch_types=[
            pltpu.VMEM((CHUNK,), jnp.int32),
            pltpu.VMEM((CHUNK,), jnp.int32),
            pltpu.VMEM((CHUNK, D_), jnp.float32),
            pltpu.VMEM((ZROWS, D_), jnp.float32),
            pltpu.VMEM_SHARED((N_NODES_, D_), jnp.float32),
            pltpu.SemaphoreType.DMA,
        ],
    )(x, row, col)


ROW_BLK = 1000


def _tc_body(p_ref, x_ref, w_ref, b_ref, o_ref):
    s = p_ref[0] + p_ref[1] + x_ref[...]
    o_ref[...] = jnp.dot(s, w_ref[...], preferred_element_type=jnp.float32) + b_ref[...]


@jax.jit
def _tc_finish(partial, x, W, b2):
    grid = N_NODES_ // ROW_BLK
    return pl.pallas_call(
        _tc_body,
        out_shape=jax.ShapeDtypeStruct((N_NODES_, D_), jnp.float32),
        grid=(grid,),
        in_specs=[
            pl.BlockSpec((NUM_CORES, ROW_BLK, D_), lambda i: (0, i, 0)),
            pl.BlockSpec((ROW_BLK, D_), lambda i: (i, 0)),
            pl.BlockSpec((D_, D_), lambda i: (0, 0)),
            pl.BlockSpec((1, D_), lambda i: (0, 0)),
        ],
        out_specs=pl.BlockSpec((ROW_BLK, D_), lambda i: (i, 0)),
    )(partial, x, W, b2)


def kernel(x, edge_index, W, b):
    ei = edge_index.astype(jnp.int32)
    row = ei[0]
    col = ei[1]
    partial = _sc_scatter(x, row, col)
    return _tc_finish(partial, x, W, b.reshape(1, D_))


# speedup vs baseline: 5.5424x; 5.5424x over previous
import functools

import jax
import jax.numpy as jnp
from jax import lax
from jax.experimental import pallas as pl
from jax.experimental.pallas import tpu as pltpu
from jax.experimental.pallas import tpu_sc as plsc

N_NODES_ = 10000
N_EDGES_ = 320000
D_ = 128

NUM_CORES = 2
NUM_SUBCORES = 16
NUM_TILES = NUM_CORES * NUM_SUBCORES
EDGES_PER_TILE = N_EDGES_ // NUM_TILES
CHUNK = 80
CHUNKS_PER_TILE = EDGES_PER_TILE // CHUNK
ROWS_PER_TILE = 624
ZCHUNK = 208
TAIL_BASE = ROWS_PER_TILE * NUM_SUBCORES
TAIL_ROWS = N_NODES_ - TAIL_BASE


def _sc_body(x_hbm, row_hbm, col_hbm, out_hbm, colv, rowv, rows_v, zbuf, acc, gsem):
    cid = lax.axis_index("c")
    sid = lax.axis_index("s")
    wid = sid * NUM_CORES + cid

    z = jnp.zeros((16,), jnp.float32)

    def _zero_body(i, _):
        for j in range(D_ // 16):
            zbuf[i, pl.ds(j * 16, 16)] = z
        return 0

    lax.fori_loop(0, ZCHUNK, _zero_body, 0)
    row_base = sid * ROWS_PER_TILE
    for k in range(ROWS_PER_TILE // ZCHUNK):
        pltpu.sync_copy(zbuf, acc.at[pl.ds(row_base + k * ZCHUNK, ZCHUNK)])

    @pl.when(sid == NUM_SUBCORES - 1)
    def _zero_tail():
        pltpu.sync_copy(zbuf.at[pl.ds(0, TAIL_ROWS)], acc.at[pl.ds(TAIL_BASE, TAIL_ROWS)])

    plsc.subcore_barrier()

    edge_base = wid * EDGES_PER_TILE

    def _chunk_body(ci, _):
        base = edge_base + ci * CHUNK
        pltpu.sync_copy(col_hbm.at[pl.ds(base, CHUNK)], colv)
        pltpu.sync_copy(row_hbm.at[pl.ds(base, CHUNK)], rowv)
        pltpu.async_copy(x_hbm.at[colv], rows_v, gsem).wait()
        pltpu.sync_copy(rows_v, acc.at[rowv], add=True)
        return 0

    lax.fori_loop(0, CHUNKS_PER_TILE, _chunk_body, 0)
    plsc.subcore_barrier()

    for k in range(ROWS_PER_TILE // ZCHUNK):
        pltpu.sync_copy(
            acc.at[pl.ds(row_base + k * ZCHUNK, ZCHUNK)],
            out_hbm.at[cid, pl.ds(row_base + k * ZCHUNK, ZCHUNK)],
        )

    @pl.when(sid == NUM_SUBCORES - 1)
    def _write_tail():
        pltpu.sync_copy(
            acc.at[pl.ds(TAIL_BASE, TAIL_ROWS)],
            out_hbm.at[cid, pl.ds(TAIL_BASE, TAIL_ROWS)],
        )


@jax.jit
def _sc_scatter(x, row, col):
    mesh = plsc.VectorSubcoreMesh(core_axis_name="c", subcore_axis_name="s")
    return pl.kernel(
        _sc_body,
        out_type=jax.ShapeDtypeStruct((NUM_CORES, N_NODES_, D_), jnp.float32),
        mesh=mesh,
        scratch_types=[
            pltpu.VMEM((CHUNK,), jnp.int32),
            pltpu.VMEM((CHUNK,), jnp.int32),
            pltpu.VMEM((CHUNK, D_), jnp.float32),
            pltpu.VMEM((ZCHUNK, D_), jnp.float32),
            pltpu.VMEM_SHARED((N_NODES_, D_), jnp.float32),
            pltpu.SemaphoreType.DMA,
        ],
    )(x, row, col)


ROW_BLK = 1000


def _tc_body(p_ref, x_ref, w_ref, b_ref, o_ref):
    s = p_ref[0] + p_ref[1] + x_ref[...]
    o_ref[...] = jnp.dot(s, w_ref[...], preferred_element_type=jnp.float32) + b_ref[...]


@jax.jit
def _tc_finish(partial, x, W, b2):
    grid = N_NODES_ // ROW_BLK
    return pl.pallas_call(
        _tc_body,
        out_shape=jax.ShapeDtypeStruct((N_NODES_, D_), jnp.float32),
        grid=(grid,),
        in_specs=[
            pl.BlockSpec((NUM_CORES, ROW_BLK, D_), lambda i: (0, i, 0)),
            pl.BlockSpec((ROW_BLK, D_), lambda i: (i, 0)),
            pl.BlockSpec((D_, D_), lambda i: (0, 0)),
            pl.BlockSpec((1, D_), lambda i: (0, 0)),
        ],
        out_specs=pl.BlockSpec((ROW_BLK, D_), lambda i: (i, 0)),
    )(partial, x, W, b2)


def kernel(x, edge_index, W, b):
    ei = edge_index.astype(jnp.int32)
    row = ei[0]
    col = ei[1]
    partial = _sc_scatter(x, row, col)
    return _tc_finish(partial, x, W, b.reshape(1, D_))


# speedup vs baseline: 12.1536x; 2.1928x over previous
import functools

import jax
import jax.numpy as jnp
from jax import lax
from jax.experimental import pallas as pl
from jax.experimental.pallas import tpu as pltpu
from jax.experimental.pallas import tpu_sc as plsc

N_NODES_ = 10000
N_EDGES_ = 320000
D_ = 128

NUM_CORES = 2
NUM_SUBCORES = 16
NUM_TILES = NUM_CORES * NUM_SUBCORES
EDGES_PER_TILE = N_EDGES_ // NUM_TILES
CHUNK = 80
CHUNKS_PER_TILE = EDGES_PER_TILE // CHUNK
ROWS_PER_TILE = 624
ZCHUNK = 208
TAIL_BASE = ROWS_PER_TILE * NUM_SUBCORES
TAIL_ROWS = N_NODES_ - TAIL_BASE


def _sc_body(x_hbm, row_hbm, col_hbm, out_hbm, colbig, rowbig,
             colv0, colv1, rowv0, rowv1, buf0, buf1, acc, gsem0, gsem1):
    cid = lax.axis_index("c")
    sid = lax.axis_index("s")
    wid = sid * NUM_CORES + cid

    z = jnp.zeros((16,), jnp.float32)

    def _zero_body(i, _):
        for j in range(D_ // 16):
            buf0[i, pl.ds(j * 16, 16)] = z
        return 0

    lax.fori_loop(0, CHUNK, _zero_body, 0)
    row_base = sid * ROWS_PER_TILE
    for k in range(ROWS_PER_TILE // CHUNK):
        pltpu.sync_copy(buf0, acc.at[pl.ds(row_base + k * CHUNK, CHUNK)])
    _zrem = ROWS_PER_TILE - (ROWS_PER_TILE // CHUNK) * CHUNK
    pltpu.sync_copy(
        buf0.at[pl.ds(0, _zrem)],
        acc.at[pl.ds(row_base + ROWS_PER_TILE - _zrem, _zrem)],
    )

    @pl.when(sid == NUM_SUBCORES - 1)
    def _zero_tail():
        pltpu.sync_copy(buf0.at[pl.ds(0, TAIL_ROWS)], acc.at[pl.ds(TAIL_BASE, TAIL_ROWS)])

    plsc.subcore_barrier()

    edge_base = wid * EDGES_PER_TILE
    pltpu.sync_copy(row_hbm.at[pl.ds(edge_base, EDGES_PER_TILE)], rowbig)
    pltpu.sync_copy(col_hbm.at[pl.ds(edge_base, EDGES_PER_TILE)], colbig)

    def _stage_idx(ci, colq, rowq):
        for j in range(CHUNK // 16):
            colq[pl.ds(j * 16, 16)] = colbig[pl.ds(ci * CHUNK + j * 16, 16)]
            rowq[pl.ds(j * 16, 16)] = rowbig[pl.ds(ci * CHUNK + j * 16, 16)]

    _stage_idx(0, colv0, rowv0)
    pltpu.async_copy(x_hbm.at[colv0], buf0, gsem0)

    def _chunk_pair(h, _):
        ci = h * 2
        _stage_idx(ci + 1, colv1, rowv1)
        pltpu.async_copy(x_hbm.at[colv1], buf1, gsem1)
        pltpu.make_async_copy(x_hbm.at[colv0], buf0, gsem0).wait()
        pltpu.sync_copy(buf0, acc.at[rowv0], add=True)
        _stage_idx(ci + 2, colv0, rowv0)
        pltpu.async_copy(x_hbm.at[colv0], buf0, gsem0)
        pltpu.make_async_copy(x_hbm.at[colv1], buf1, gsem1).wait()
        pltpu.sync_copy(buf1, acc.at[rowv1], add=True)
        return 0

    lax.fori_loop(0, (CHUNKS_PER_TILE - 1) // 2, _chunk_pair, 0)
    pltpu.make_async_copy(x_hbm.at[colv0], buf0, gsem0).wait()
    pltpu.sync_copy(buf0, acc.at[rowv0], add=True)
    plsc.subcore_barrier()

    for k in range(ROWS_PER_TILE // ZCHUNK):
        pltpu.sync_copy(
            acc.at[pl.ds(row_base + k * ZCHUNK, ZCHUNK)],
            out_hbm.at[cid, pl.ds(row_base + k * ZCHUNK, ZCHUNK)],
        )

    @pl.when(sid == NUM_SUBCORES - 1)
    def _write_tail():
        pltpu.sync_copy(
            acc.at[pl.ds(TAIL_BASE, TAIL_ROWS)],
            out_hbm.at[cid, pl.ds(TAIL_BASE, TAIL_ROWS)],
        )


@jax.jit
def _sc_scatter(x, row, col):
    mesh = plsc.VectorSubcoreMesh(core_axis_name="c", subcore_axis_name="s")
    return pl.kernel(
        _sc_body,
        out_type=jax.ShapeDtypeStruct((NUM_CORES, N_NODES_, D_), jnp.float32),
        mesh=mesh,
        scratch_types=[
            pltpu.VMEM((EDGES_PER_TILE,), jnp.int32),
            pltpu.VMEM((EDGES_PER_TILE,), jnp.int32),
            pltpu.VMEM((CHUNK,), jnp.int32),
            pltpu.VMEM((CHUNK,), jnp.int32),
            pltpu.VMEM((CHUNK,), jnp.int32),
            pltpu.VMEM((CHUNK,), jnp.int32),
            pltpu.VMEM((CHUNK, D_), jnp.float32),
            pltpu.VMEM((CHUNK, D_), jnp.float32),
            pltpu.VMEM_SHARED((N_NODES_, D_), jnp.float32),
            pltpu.SemaphoreType.DMA,
            pltpu.SemaphoreType.DMA,
        ],
    )(x, row, col)


ROW_BLK = 1000


def _tc_body(p_ref, x_ref, w_ref, b_ref, o_ref):
    s = p_ref[0] + p_ref[1] + x_ref[...]
    o_ref[...] = jnp.dot(s, w_ref[...], preferred_element_type=jnp.float32) + b_ref[...]


@jax.jit
def _tc_finish(partial, x, W, b2):
    grid = N_NODES_ // ROW_BLK
    return pl.pallas_call(
        _tc_body,
        out_shape=jax.ShapeDtypeStruct((N_NODES_, D_), jnp.float32),
        grid=(grid,),
        in_specs=[
            pl.BlockSpec((NUM_CORES, ROW_BLK, D_), lambda i: (0, i, 0)),
            pl.BlockSpec((ROW_BLK, D_), lambda i: (i, 0)),
            pl.BlockSpec((D_, D_), lambda i: (0, 0)),
            pl.BlockSpec((1, D_), lambda i: (0, 0)),
        ],
        out_specs=pl.BlockSpec((ROW_BLK, D_), lambda i: (i, 0)),
    )(partial, x, W, b2)


def kernel(x, edge_index, W, b):
    ei = edge_index.astype(jnp.int32)
    row = ei[0]
    col = ei[1]
    partial = _sc_scatter(x, row, col)
    return _tc_finish(partial, x, W, b.reshape(1, D_))
